# baseline (speedup 1.0000x reference)
import functools

import jax
import jax.numpy as jnp
from jax import lax
from jax.experimental import pallas as pl
from jax.experimental.pallas import tpu as pltpu
from jax.experimental.pallas import tpu_sc as plsc

NC, NS = 2, 16
NW = NC * NS
R = 16384
S = 50
D = 64
V = 100
NI = R // NW
L = 16
G = 8
PW = D + 1
CW = V * PW + 1
NI2 = NI // 2
NB2 = NI2 // L

_mesh = plsc.VectorSubcoreMesh(core_axis_name="c", subcore_axis_name="s")


@functools.partial(
    pl.kernel,
    out_type=jax.ShapeDtypeStruct((S, D, R), jnp.float32),
    mesh=_mesh,
    scratch_types=[
        pltpu.VMEM((G * CW,), jnp.float32),
        pltpu.VMEM((S, NI), jnp.int32),
        pltpu.VMEM((2 * D, NI2), jnp.float32),
        pltpu.SemaphoreType.DMA,
        pltpu.SemaphoreType.DMA,
    ],
    compiler_params=pltpu.CompilerParams(
        use_tc_tiling_on_sc=True, needs_layout_passes=False
    ),
)
def _embed(idxT_hbm, tab_hbm, out_hbm, tab_v, idx_v, obuf, tsem, ssem):
    wid = lax.axis_index("s") * NC + lax.axis_index("c")
    i0 = wid * NI

    pltpu.async_copy(tab_hbm, tab_v, tsem).wait()
    pltpu.async_copy(idxT_hbm.at[:, pl.ds(i0, NI)], idx_v, tsem).wait()

    lane_off = (lax.iota(jnp.int32, L) & (G - 1)) * CW

    def wait_store_one():
        pltpu.make_async_copy(
            obuf.at[pl.ds(0, D)], out_hbm.at[0, :, pl.ds(i0, NI2)], ssem
        ).wait()

    @pl.loop(0, 2 * S)
    def _main(u):
        s = u // 2
        h = u % 2
        base = h * D

        @pl.when(u >= 2)
        def _():
            wait_store_one()

        for ib in range(NB2):
            sl = pl.ds(ib * L, L)
            a = lane_off + idx_v[s, pl.ds(h * NI2 + ib * L, L)] * PW

            @plsc.parallel_loop(0, D, unroll=16)
            def _gather(d):
                obuf[base + d, sl] = plsc.load_gather(tab_v, [a + d])

        pltpu.async_copy(
            obuf.at[pl.ds(base, D)],
            out_hbm.at[s, :, pl.ds(i0 + h * NI2, NI2)],
            ssem,
        )

    wait_store_one()
    wait_store_one()


def kernel(product_id, product_embed_weight):
    idxT = product_id.T
    w65 = jnp.pad(product_embed_weight, ((0, 0), (0, 1)))
    tab = jnp.tile(jnp.pad(w65.reshape(-1), (0, 1)), G)
    out = _embed(idxT, tab)
    return jnp.transpose(out, (2, 0, 1))


# speedup vs baseline: 1.0593x; 1.0593x over previous
import functools

import jax
import jax.numpy as jnp
from jax import lax
from jax.experimental import pallas as pl
from jax.experimental.pallas import tpu as pltpu
from jax.experimental.pallas import tpu_sc as plsc

NC, NS = 2, 16
NW = NC * NS
R = 16384
S = 50
D = 64
V = 100
NI = R // NW
L = 16
G = 8
PW = D + 1
CW = V * PW + 1
NI2 = NI // 2
NB2 = NI2 // L

_mesh = plsc.VectorSubcoreMesh(core_axis_name="c", subcore_axis_name="s")


@functools.partial(
    pl.kernel,
    out_type=jax.ShapeDtypeStruct((S, D, R), jnp.float32),
    mesh=_mesh,
    scratch_types=[
        pltpu.VMEM((G * CW,), jnp.float32),
        pltpu.VMEM((S, NI), jnp.int32),
        pltpu.VMEM((2 * D, NI2), jnp.float32),
        pltpu.SemaphoreType.DMA,
        pltpu.SemaphoreType.DMA,
    ],
    compiler_params=pltpu.CompilerParams(
        use_tc_tiling_on_sc=True, needs_layout_passes=False
    ),
)
def _embed(idxT_hbm, tab_hbm, out_hbm, tab_v, idx_v, obuf, tsem, ssem):
    wid = lax.axis_index("s") * NC + lax.axis_index("c")
    i0 = wid * NI

    pltpu.async_copy(tab_hbm, tab_v, tsem).wait()
    pltpu.async_copy(idxT_hbm.at[:, pl.ds(i0, NI)], idx_v, tsem).wait()

    lane_off = (lax.iota(jnp.int32, L) & (G - 1)) * CW

    def wait_store_one():
        pltpu.make_async_copy(
            obuf.at[pl.ds(0, D)], out_hbm.at[0, :, pl.ds(i0, NI2)], ssem
        ).wait()

    @pl.loop(0, 2 * S)
    def _main(u):
        s = u // 2
        h = u % 2
        base = h * D

        @pl.when(u >= 2)
        def _():
            wait_store_one()

        for ib in range(NB2):
            sl = pl.ds(ib * L, L)
            a = lane_off + idx_v[s, pl.ds(h * NI2 + ib * L, L)] * PW

            @plsc.parallel_loop(0, D, unroll=8)
            def _gather(d):
                obuf[base + d, sl] = plsc.load_gather(tab_v, [a + d])

        pltpu.async_copy(
            obuf.at[pl.ds(base, D)],
            out_hbm.at[s, :, pl.ds(i0 + h * NI2, NI2)],
            ssem,
        )

    wait_store_one()
    wait_store_one()


def kernel(product_id, product_embed_weight):
    idxT = product_id.T
    w65 = jnp.pad(product_embed_weight, ((0, 0), (0, 1)))
    tab = jnp.tile(jnp.pad(w65.reshape(-1), (0, 1)), G)
    out = _embed(idxT, tab)
    return jnp.transpose(out, (2, 0, 1))
